# baseline (speedup 1.0000x reference)
import jax
import jax.numpy as jnp
from jax.experimental import pallas as pl
from jax.experimental.pallas import tpu as pltpu


_NB = 512


def _fused_body(x_ref, wc_ref, bc_ref, w1_ref, b1_ref, w2_ref, b2_ref, o_ref):
    nb = o_ref.shape[0]
    h = None
    for p in range(12):
        xs = x_ref[:, 0, 2 * p:2 * p + 6, :].astype(jnp.bfloat16)
        lhs = xs.reshape(nb, 168)
        acc = jnp.dot(lhs, wc_ref[...], preferred_element_type=jnp.float32)
        z = jnp.maximum(jnp.maximum(acc[:, 0:384], acc[:, 384:768]),
                        jnp.maximum(acc[:, 768:1152], acc[:, 1152:1536]))
        z = jnp.maximum(z + bc_ref[...], 0.0).astype(jnp.bfloat16)
        part = jnp.dot(z, w1_ref[p], preferred_element_type=jnp.float32)
        h = part if h is None else h + part
    h = jnp.maximum(h + b1_ref[...], 0.0)
    o_ref[...] = jnp.dot(h, w2_ref[...],
                         preferred_element_type=jnp.float32) + b2_ref[...]


def _band_weights(conv_w):
    kk = jnp.arange(5)
    j = jnp.arange(28).reshape(28, 1, 1, 1)
    dx = jnp.arange(2).reshape(1, 2, 1, 1)
    pw = jnp.arange(12).reshape(1, 1, 12, 1)
    b = (j - 2 * pw - dx == kk).astype(jnp.float32).reshape(672, 5)
    w5 = conv_w[:, 0]
    c5 = jnp.einsum("bq,ckq->kbc", b, w5)
    c5b = c5.reshape(5, 1, 28, 1, 2, 12, 32)
    s = jnp.arange(6).reshape(6, 1, 1)
    dy = jnp.arange(2).reshape(1, 2, 1)
    a = (s - dy == kk).astype(jnp.float32)
    ab = a.transpose(2, 0, 1).reshape(5, 6, 1, 2, 1, 1, 1)
    wc = jnp.sum(ab * c5b, axis=0)
    return wc.reshape(168, 1536).astype(jnp.bfloat16)


def kernel(x, conv_w, conv_b, fc1_w, fc1_b, fc2_w, fc2_b):
    n = x.shape[0]
    nb = _NB if n >= _NB else 8
    n_pad = -(-n // nb) * nb
    x3 = x
    if n_pad != n:
        x3 = jnp.pad(x3, ((0, n_pad - n), (0, 0), (0, 0), (0, 0)))

    wc = _band_weights(conv_w)
    bc = jnp.tile(conv_b, 12).reshape(1, 384)
    w1r = (fc1_w.astype(jnp.bfloat16)
           .reshape(128, 32, 12, 12)
           .transpose(2, 3, 1, 0)
           .reshape(12, 384, 128))
    b1 = fc1_b.reshape(1, 128)
    w2p = jnp.zeros((128, 128), jnp.float32).at[:, :10].set(fc2_w.T)
    b2p = jnp.zeros((1, 128), jnp.float32).at[0, :10].set(fc2_b)

    bidx = lambda i: (i, 0, 0, 0)
    zero2 = lambda i: (0, 0)
    zero3 = lambda i: (0, 0, 0)

    out = pl.pallas_call(
        _fused_body,
        out_shape=jax.ShapeDtypeStruct((n_pad, 128), jnp.float32),
        grid=(n_pad // nb,),
        in_specs=[
            pl.BlockSpec((nb, 1, 28, 28), bidx),
            pl.BlockSpec((168, 1536), zero2),
            pl.BlockSpec((1, 384), zero2),
            pl.BlockSpec((12, 384, 128), zero3),
            pl.BlockSpec((1, 128), zero2),
            pl.BlockSpec((128, 128), zero2),
            pl.BlockSpec((1, 128), zero2),
        ],
        out_specs=pl.BlockSpec((nb, 128), lambda i: (i, 0)),
        compiler_params=pltpu.CompilerParams(
            dimension_semantics=("arbitrary",)),
        name="fused_convnet",
    )(x3, wc, bc, w1r, b1, w2p, b2p)
    return out[:n, :10]


# speedup vs baseline: 1.6681x; 1.6681x over previous
import jax
import jax.numpy as jnp
from jax.experimental import pallas as pl
from jax.experimental.pallas import tpu as pltpu


_NB = 512


def _fused_body(x_ref, wc_ref, bc_ref, w1_ref, b1_ref, w2_ref, b2_ref, o_ref):
    nb = o_ref.shape[0]
    xf = x_ref[...].astype(jnp.bfloat16).reshape(nb, 784)
    h = None
    for p in range(12):
        lhs = xf[:, 56 * p:56 * p + 168]
        acc = jnp.dot(lhs, wc_ref[...], preferred_element_type=jnp.float32)
        z = jnp.maximum(jnp.maximum(acc[:, 0:384], acc[:, 384:768]),
                        jnp.maximum(acc[:, 768:1152], acc[:, 1152:1536]))
        z = jnp.maximum(z + bc_ref[...], 0.0).astype(jnp.bfloat16)
        part = jnp.dot(z, w1_ref[p], preferred_element_type=jnp.float32)
        h = part if h is None else h + part
    h = jnp.maximum(h + b1_ref[...], 0.0)
    o_ref[...] = jnp.dot(h, w2_ref[...],
                         preferred_element_type=jnp.float32) + b2_ref[...]


def _band_weights(conv_w):
    kk = jnp.arange(5)
    j = jnp.arange(28).reshape(28, 1, 1, 1)
    dx = jnp.arange(2).reshape(1, 2, 1, 1)
    pw = jnp.arange(12).reshape(1, 1, 12, 1)
    b = (j - 2 * pw - dx == kk).astype(jnp.float32).reshape(672, 5)
    w5 = conv_w[:, 0]
    c5 = jnp.einsum("bq,ckq->kbc", b, w5)
    c5b = c5.reshape(5, 1, 28, 1, 2, 12, 32)
    s = jnp.arange(6).reshape(6, 1, 1)
    dy = jnp.arange(2).reshape(1, 2, 1)
    a = (s - dy == kk).astype(jnp.float32)
    ab = a.transpose(2, 0, 1).reshape(5, 6, 1, 2, 1, 1, 1)
    wc = jnp.sum(ab * c5b, axis=0)
    return wc.reshape(168, 1536).astype(jnp.bfloat16)


def kernel(x, conv_w, conv_b, fc1_w, fc1_b, fc2_w, fc2_b):
    n = x.shape[0]
    nb = _NB if n >= _NB else 8
    n_pad = -(-n // nb) * nb
    x3 = x.reshape(n, 28, 28)
    if n_pad != n:
        x3 = jnp.pad(x3, ((0, n_pad - n), (0, 0), (0, 0)))

    wc = _band_weights(conv_w)
    bc = jnp.tile(conv_b, 12).reshape(1, 384)
    w1r = (fc1_w.astype(jnp.bfloat16)
           .reshape(128, 32, 12, 12)
           .transpose(2, 3, 1, 0)
           .reshape(12, 384, 128))
    b1 = fc1_b.reshape(1, 128)
    w2p = jnp.zeros((128, 128), jnp.float32).at[:, :10].set(fc2_w.T)
    b2p = jnp.zeros((1, 128), jnp.float32).at[0, :10].set(fc2_b)

    bidx = lambda i: (i, 0, 0)
    zero2 = lambda i: (0, 0)
    zero3 = lambda i: (0, 0, 0)

    out = pl.pallas_call(
        _fused_body,
        out_shape=jax.ShapeDtypeStruct((n_pad, 128), jnp.float32),
        grid=(n_pad // nb,),
        in_specs=[
            pl.BlockSpec((nb, 28, 28), bidx),
            pl.BlockSpec((168, 1536), zero2),
            pl.BlockSpec((1, 384), zero2),
            pl.BlockSpec((12, 384, 128), zero3),
            pl.BlockSpec((1, 128), zero2),
            pl.BlockSpec((128, 128), zero2),
            pl.BlockSpec((1, 128), zero2),
        ],
        out_specs=pl.BlockSpec((nb, 128), lambda i: (i, 0)),
        compiler_params=pltpu.CompilerParams(
            dimension_semantics=("arbitrary",)),
        name="fused_convnet",
    )(x3, wc, bc, w1r, b1, w2p, b2p)
    return out[:n, :10]


# speedup vs baseline: 1.7804x; 1.0673x over previous
import jax
import jax.numpy as jnp
from jax.experimental import pallas as pl
from jax.experimental.pallas import tpu as pltpu


_NB = 512


def _fused_body(x_ref, wc_ref, bc_ref, w1_ref, b1_ref, w2_ref, b2_ref, o_ref):
    nb = o_ref.shape[0]
    xv = x_ref[...].astype(jnp.bfloat16)
    xf = jnp.pad(xv, ((0, 0), (0, 0), (0, 4))).reshape(nb, 896)
    h = None
    for p in range(12):
        lhs = xf[:, 64 * p:64 * p + 192]
        acc = jnp.dot(lhs, wc_ref[...], preferred_element_type=jnp.float32)
        ab = acc.astype(jnp.bfloat16)
        z = jnp.maximum(jnp.maximum(ab[:, 0:384], ab[:, 384:768]),
                        jnp.maximum(ab[:, 768:1152], ab[:, 1152:1536]))
        z = jnp.maximum(z + bc_ref[...], 0.0)
        part = jnp.dot(z, w1_ref[p], preferred_element_type=jnp.float32)
        h = part if h is None else h + part
    h = jnp.maximum(h + b1_ref[...], 0.0)
    o_ref[...] = jnp.dot(h, w2_ref[...],
                         preferred_element_type=jnp.float32) + b2_ref[...]


def _band_weights(conv_w):
    kk = jnp.arange(5)
    j = jnp.arange(32).reshape(32, 1, 1, 1)
    dx = jnp.arange(2).reshape(1, 2, 1, 1)
    pw = jnp.arange(12).reshape(1, 1, 12, 1)
    b = (j - 2 * pw - dx == kk).astype(jnp.float32).reshape(768, 5)
    w5 = conv_w[:, 0]
    c5 = jnp.einsum("bq,ckq->kbc", b, w5)
    c5b = c5.reshape(5, 1, 32, 1, 2, 12, 32)
    s = jnp.arange(6).reshape(6, 1, 1)
    dy = jnp.arange(2).reshape(1, 2, 1)
    a = (s - dy == kk).astype(jnp.float32)
    ab = a.transpose(2, 0, 1).reshape(5, 6, 1, 2, 1, 1, 1)
    wc = jnp.sum(ab * c5b, axis=0)
    return wc.reshape(192, 1536).astype(jnp.bfloat16)


def kernel(x, conv_w, conv_b, fc1_w, fc1_b, fc2_w, fc2_b):
    n = x.shape[0]
    nb = _NB if n >= _NB else 8
    n_pad = -(-n // nb) * nb
    x3 = x.reshape(n, 28, 28)
    if n_pad != n:
        x3 = jnp.pad(x3, ((0, n_pad - n), (0, 0), (0, 0)))

    wc = _band_weights(conv_w)
    bc = jnp.tile(conv_b, 12).reshape(1, 384).astype(jnp.bfloat16)
    w1r = (fc1_w.astype(jnp.bfloat16)
           .reshape(128, 32, 12, 12)
           .transpose(2, 3, 1, 0)
           .reshape(12, 384, 128))
    b1 = fc1_b.reshape(1, 128)
    w2p = jnp.zeros((128, 128), jnp.float32).at[:, :10].set(fc2_w.T)
    b2p = jnp.zeros((1, 128), jnp.float32).at[0, :10].set(fc2_b)

    bidx = lambda i: (i, 0, 0)
    zero2 = lambda i: (0, 0)
    zero3 = lambda i: (0, 0, 0)

    out = pl.pallas_call(
        _fused_body,
        out_shape=jax.ShapeDtypeStruct((n_pad, 128), jnp.float32),
        grid=(n_pad // nb,),
        in_specs=[
            pl.BlockSpec((nb, 28, 28), bidx),
            pl.BlockSpec((192, 1536), zero2),
            pl.BlockSpec((1, 384), zero2),
            pl.BlockSpec((12, 384, 128), zero3),
            pl.BlockSpec((1, 128), zero2),
            pl.BlockSpec((128, 128), zero2),
            pl.BlockSpec((1, 128), zero2),
        ],
        out_specs=pl.BlockSpec((nb, 128), lambda i: (i, 0)),
        compiler_params=pltpu.CompilerParams(
            dimension_semantics=("arbitrary",)),
        name="fused_convnet",
    )(x3, wc, bc, w1r, b1, w2p, b2p)
    return out[:n, :10]


# speedup vs baseline: 1.8075x; 1.0152x over previous
import jax
import jax.numpy as jnp
from jax.experimental import pallas as pl
from jax.experimental.pallas import tpu as pltpu


_NB = 1024


def _fused_body(x_ref, wc_ref, bc_ref, w1_ref, b1_ref, w2_ref, b2_ref, o_ref):
    nb = o_ref.shape[0]
    xv = x_ref[...].astype(jnp.bfloat16)
    xf = jnp.pad(xv, ((0, 0), (0, 0), (0, 4))).reshape(nb, 896)
    h = None
    for p in range(12):
        lhs = xf[:, 64 * p:64 * p + 192]
        acc = jnp.dot(lhs, wc_ref[...], preferred_element_type=jnp.float32)
        ab = acc.astype(jnp.bfloat16)
        z = jnp.maximum(jnp.maximum(ab[:, 0:384], ab[:, 384:768]),
                        jnp.maximum(ab[:, 768:1152], ab[:, 1152:1536]))
        z = jnp.maximum(z + bc_ref[...], 0.0)
        part = jnp.dot(z, w1_ref[p], preferred_element_type=jnp.float32)
        h = part if h is None else h + part
    h = jnp.maximum(h + b1_ref[...], 0.0)
    o_ref[...] = jnp.dot(h, w2_ref[...],
                         preferred_element_type=jnp.float32) + b2_ref[...]


def _band_weights(conv_w):
    kk = jnp.arange(5)
    j = jnp.arange(32).reshape(32, 1, 1, 1)
    dx = jnp.arange(2).reshape(1, 2, 1, 1)
    pw = jnp.arange(12).reshape(1, 1, 12, 1)
    b = (j - 2 * pw - dx == kk).astype(jnp.float32).reshape(768, 5)
    w5 = conv_w[:, 0]
    c5 = jnp.einsum("bq,ckq->kbc", b, w5)
    c5b = c5.reshape(5, 1, 32, 1, 2, 12, 32)
    s = jnp.arange(6).reshape(6, 1, 1)
    dy = jnp.arange(2).reshape(1, 2, 1)
    a = (s - dy == kk).astype(jnp.float32)
    ab = a.transpose(2, 0, 1).reshape(5, 6, 1, 2, 1, 1, 1)
    wc = jnp.sum(ab * c5b, axis=0)
    return wc.reshape(192, 1536).astype(jnp.bfloat16)


def kernel(x, conv_w, conv_b, fc1_w, fc1_b, fc2_w, fc2_b):
    n = x.shape[0]
    nb = _NB if n >= _NB else 8
    n_pad = -(-n // nb) * nb
    x3 = x.reshape(n, 28, 28)
    if n_pad != n:
        x3 = jnp.pad(x3, ((0, n_pad - n), (0, 0), (0, 0)))

    wc = _band_weights(conv_w)
    bc = jnp.tile(conv_b, 12).reshape(1, 384).astype(jnp.bfloat16)
    w1r = (fc1_w.astype(jnp.bfloat16)
           .reshape(128, 32, 12, 12)
           .transpose(2, 3, 1, 0)
           .reshape(12, 384, 128))
    b1 = fc1_b.reshape(1, 128)
    w2p = jnp.zeros((128, 128), jnp.float32).at[:, :10].set(fc2_w.T)
    b2p = jnp.zeros((1, 128), jnp.float32).at[0, :10].set(fc2_b)

    bidx = lambda i: (i, 0, 0)
    zero2 = lambda i: (0, 0)
    zero3 = lambda i: (0, 0, 0)

    out = pl.pallas_call(
        _fused_body,
        out_shape=jax.ShapeDtypeStruct((n_pad, 128), jnp.float32),
        grid=(n_pad // nb,),
        in_specs=[
            pl.BlockSpec((nb, 28, 28), bidx),
            pl.BlockSpec((192, 1536), zero2),
            pl.BlockSpec((1, 384), zero2),
            pl.BlockSpec((12, 384, 128), zero3),
            pl.BlockSpec((1, 128), zero2),
            pl.BlockSpec((128, 128), zero2),
            pl.BlockSpec((1, 128), zero2),
        ],
        out_specs=pl.BlockSpec((nb, 128), lambda i: (i, 0)),
        compiler_params=pltpu.CompilerParams(
            dimension_semantics=("arbitrary",)),
        name="fused_convnet",
    )(x3, wc, bc, w1r, b1, w2p, b2p)
    return out[:n, :10]
